# baseline (speedup 1.0000x reference)
import dataclasses
import functools

import jax
import jax.numpy as jnp
from jax import lax
from jax.experimental import pallas as pl
from jax.experimental.pallas import tpu as pltpu
from jax.experimental.pallas import tpu_sc as plsc

NEG_SLOPE = 0.2
LANES = 16
CHUNK = 64
NC, NS = 2, 16
NW = NC * NS


def _prep_body(x_ref, w_ref, asrc_ref, adst_ref, h_ref, sd_ref):
    n = x_ref.shape[0]
    n_ext = sd_ref.shape[1]
    h = jnp.dot(x_ref[...], w_ref[...], preferred_element_type=jnp.float32)
    h_ref[...] = h
    s = jnp.dot(h, asrc_ref[...].reshape(-1, 1),
                preferred_element_type=jnp.float32)[:, 0]
    d = jnp.dot(h, adst_ref[...].reshape(-1, 1),
                preferred_element_type=jnp.float32)[:, 0]
    c = jnp.max(s) + jnp.max(d)
    c = jnp.where(c > 0, c, NEG_SLOPE * c)
    pad = jnp.zeros((n_ext - n,), jnp.float32)
    s_ext = jnp.concatenate([s, pad])
    d_ext = jnp.concatenate([d, pad])
    idx = lax.broadcasted_iota(jnp.int32, (n_ext,), 0)
    s_ext = jnp.where(idx == n, c, s_ext)
    sd_ref[...] = jnp.stack([s_ext, d_ext])


def _final_body(u_ref, d_ref, bias_ref, out_ref):
    num = u_ref[0] + u_ref[1]
    den = jnp.sum(d_ref[...], axis=0)
    out_ref[...] = jnp.tanh(num / den[:, None] + bias_ref[...][None, :])


def _sc_body(n_nodes, n_total, per_w, n_chunks,
             h_hbm, sd_hbm, src_hbm, dst_hbm, zero_hbm, out_hbm, den_hbm,
             u_sh, s_v, d_v, denl, srcv, tv0, tv1, hb0, hb1, eev,
             sem_g, su0, su1):
    cid = lax.axis_index("c")
    sid = lax.axis_index("s")
    wid = sid * NC + cid
    rows_per_tile = (n_nodes // NS) // 8 * 8
    rem_base = rows_per_tile * NS
    rem = n_nodes - rem_base

    off = pl.multiple_of(sid * rows_per_tile, 8)
    pltpu.sync_copy(zero_hbm.at[pl.ds(off, rows_per_tile)],
                    u_sh.at[pl.ds(off, rows_per_tile)])
    if rem:
        @pl.when(sid == 0)
        def _():
            pltpu.sync_copy(zero_hbm.at[pl.ds(rem_base, rem)],
                            u_sh.at[pl.ds(rem_base, rem)])
    pltpu.sync_copy(sd_hbm.at[0], s_v)
    pltpu.sync_copy(sd_hbm.at[1], d_v)

    def zden(j, carry):
        denl[pl.ds(j * LANES, LANES)] = jnp.zeros((LANES,), jnp.float32)
        return carry
    lax.fori_loop(0, (denl.shape[0] + LANES - 1) // LANES, zden, 0)
    plsc.subcore_barrier()

    cvec = plsc.load_gather(s_v, [jnp.full((LANES,), n_nodes, jnp.int32)])
    lane0 = lax.iota(jnp.int32, LANES) == 0
    dsts = (tv0, tv1)
    hbufs = (hb0, hb1)
    sus = (su0, su1)

    def do_chunk(c, b, wait_prev):
        dstv = dsts[b]
        hrows = hbufs[b]
        base = pl.multiple_of(wid * per_w + c * CHUNK, CHUNK)
        pltpu.sync_copy(src_hbm.at[pl.ds(base, CHUNK)], srcv)
        pltpu.sync_copy(dst_hbm.at[pl.ds(base, CHUNK)], dstv)
        pltpu.async_copy(h_hbm.at[srcv], hrows, sem_g).wait()

        def group_body(g, carry2):
            sidx = srcv[pl.ds(g * LANES, LANES)]
            didx = dstv[pl.ds(g * LANES, LANES)]
            e = plsc.load_gather(s_v, [sidx]) + plsc.load_gather(d_v, [didx])
            e = jnp.where(e > 0, e, NEG_SLOPE * e) - cvec
            gi = base + g * LANES + lax.iota(jnp.int32, LANES)
            eev[pl.ds(g * LANES, LANES)] = jnp.where(
                gi < n_total, jnp.exp(e), 0.0)
            return carry2

        lax.fori_loop(0, CHUNK // LANES, group_body, 0)

        def edge_body(k4, carry2):
            kb = k4 * 4
            for u in range(4):
                k = kb + u
                kidx = jnp.zeros((LANES,), jnp.int32) + k
                spl = plsc.load_gather(eev, [kidx])
                for cc in range(8):
                    hrows[k, pl.ds(cc * LANES, LANES)] = (
                        hrows[k, pl.ds(cc * LANES, LANES)] * spl)
                dsp = plsc.load_gather(dstv, [kidx])
                dcur = plsc.load_gather(denl, [dsp])
                plsc.store_scatter(denl, [dsp], dcur + spl, mask=lane0)
            return carry2

        lax.fori_loop(0, CHUNK // 4, edge_body, 0)
        if wait_prev:
            pltpu.make_async_copy(hbufs[1 - b], u_sh.at[dsts[1 - b]],
                                  sus[1 - b]).wait()
        pltpu.async_copy(hrows, u_sh.at[dstv], sus[b], add=True)

    do_chunk(0, 0, False)
    do_chunk(1, 1, True)

    def pair_body(it, carry):
        do_chunk(2 * it, 0, True)
        do_chunk(2 * it + 1, 1, True)
        return carry

    lax.fori_loop(1, n_chunks // 2, pair_body, 0)
    pltpu.make_async_copy(hbufs[1], u_sh.at[dsts[1]], sus[1]).wait()
    pltpu.sync_copy(denl, den_hbm.at[wid])
    plsc.subcore_barrier()

    @pl.when(sid == 0)
    def _():
        pltpu.sync_copy(u_sh, out_hbm.at[cid])


def kernel(x, edge_index, W, att_src, att_dst, bias):
    n, _ = x.shape
    dout = W.shape[1]
    e = edge_index.shape[1]
    n_total = e + n
    n_chunks = -(-n_total // (NW * CHUNK))
    n_chunks = -(-n_chunks // 2) * 2
    per_w = n_chunks * CHUNK
    epad = NW * per_w
    n_ext = n + LANES

    loops = jnp.arange(n, dtype=edge_index.dtype)
    padz = jnp.zeros((epad - n_total,), edge_index.dtype)
    src = jnp.concatenate([edge_index[0], loops, padz])
    dst = jnp.concatenate([edge_index[1], loops, padz])

    h, sd = pl.pallas_call(
        _prep_body,
        out_shape=(
            jax.ShapeDtypeStruct((n, dout), jnp.float32),
            jax.ShapeDtypeStruct((2, n_ext), jnp.float32),
        ),
    )(x, W, att_src, att_dst)

    zero = jnp.zeros((n, dout), jnp.float32)

    mesh = plsc.VectorSubcoreMesh(
        core_axis_name="c", subcore_axis_name="s",
        num_cores=NC, num_subcores=NS)
    cp = pltpu.CompilerParams()
    if "needs_layout_passes" in pltpu.CompilerParams.__dataclass_fields__:
        cp = dataclasses.replace(cp, needs_layout_passes=False)
    sc_fn = pl.kernel(
        functools.partial(_sc_body, n, n_total, per_w, n_chunks),
        out_type=(jax.ShapeDtypeStruct((NC, n, dout), jnp.float32),
                  jax.ShapeDtypeStruct((NW, n), jnp.float32)),
        mesh=mesh,
        compiler_params=cp,
        scratch_types=[
            pltpu.VMEM_SHARED((n, dout), jnp.float32),
            pltpu.VMEM((n_ext,), jnp.float32),
            pltpu.VMEM((n_ext,), jnp.float32),
            pltpu.VMEM((n,), jnp.float32),
            pltpu.VMEM((CHUNK,), jnp.int32),
            pltpu.VMEM((CHUNK,), jnp.int32),
            pltpu.VMEM((CHUNK,), jnp.int32),
            pltpu.VMEM((CHUNK, 128), jnp.float32),
            pltpu.VMEM((CHUNK, 128), jnp.float32),
            pltpu.VMEM((CHUNK,), jnp.float32),
            pltpu.SemaphoreType.DMA,
            pltpu.SemaphoreType.DMA,
            pltpu.SemaphoreType.DMA,
        ],
    )
    upart, dpart = sc_fn(h, sd, src, dst, zero)

    out = pl.pallas_call(
        _final_body,
        out_shape=jax.ShapeDtypeStruct((n, dout), jnp.float32),
    )(upart, dpart, bias)
    return out


# speedup vs baseline: 1.7868x; 1.7868x over previous
import dataclasses
import functools

import jax
import jax.numpy as jnp
from jax import lax
from jax.experimental import pallas as pl
from jax.experimental.pallas import tpu as pltpu
from jax.experimental.pallas import tpu_sc as plsc

NEG_SLOPE = 0.2
LANES = 16
CHUNK = 64
NC, NS = 2, 16
NW = NC * NS
RING = 6


def _prep_body(x_ref, w_ref, asrc_ref, adst_ref, h_ref, s_ref, d_ref):
    n = x_ref.shape[0]
    n_ext = s_ref.shape[0]
    h = jnp.dot(x_ref[...], w_ref[...], preferred_element_type=jnp.float32)
    h_ref[...] = h
    s = jnp.dot(h, asrc_ref[...].reshape(-1, 1),
                preferred_element_type=jnp.float32)[:, 0]
    d = jnp.dot(h, adst_ref[...].reshape(-1, 1),
                preferred_element_type=jnp.float32)[:, 0]
    c = jnp.max(s) + jnp.max(d)
    c = jnp.where(c > 0, c, NEG_SLOPE * c)
    pad = jnp.zeros((n_ext - n,), jnp.float32)
    s_ext = jnp.concatenate([s, pad])
    d_ext = jnp.concatenate([d, pad])
    idx = lax.broadcasted_iota(jnp.int32, (n_ext,), 0)
    s_ref[...] = jnp.where(idx == n, c, s_ext)
    d_ref[...] = d_ext


def _final_body(u_ref, d_ref, bias_ref, out_ref):
    num = u_ref[0] + u_ref[1]
    den = d_ref[0] + d_ref[1]
    out_ref[...] = jnp.tanh(num / den[:, None] + bias_ref[...][None, :])


def _sc_body(n_nodes, n_total, per_w, n_chunks,
             h_hbm, s_hbm, d_hbm, src_hbm, dst_hbm, zero_hbm, zden_hbm,
             out_hbm, den_hbm,
             u_sh, den_sh, s_v, d_v,
             sv0, sv1, tv0, tv1, tv2, hb0, hb1, hb2, ee0, ee1,
             sg0, sg1, sg2, su0, su1, su2):
    cid = lax.axis_index("c")
    sid = lax.axis_index("s")
    wid = sid * NC + cid
    rows_per_tile = (n_nodes // NS) // 8 * 8
    rem_base = rows_per_tile * NS
    rem = n_nodes - rem_base
    srcs = (sv0, sv1)
    dsts = (tv0, tv1, tv2)
    hbufs = (hb0, hb1, hb2)
    eevs = (ee0, ee1)
    sgs = (sg0, sg1, sg2)
    sus = (su0, su1, su2)

    off = pl.multiple_of(sid * rows_per_tile, 8)
    pltpu.sync_copy(zero_hbm.at[pl.ds(off, rows_per_tile)],
                    u_sh.at[pl.ds(off, rows_per_tile)])
    if rem:
        @pl.when(sid == 0)
        def _():
            pltpu.sync_copy(zero_hbm.at[pl.ds(rem_base, rem)],
                            u_sh.at[pl.ds(rem_base, rem)])

    @pl.when(sid == 0)
    def _():
        pltpu.sync_copy(zden_hbm, den_sh)

    pltpu.sync_copy(s_hbm, s_v)
    pltpu.sync_copy(d_hbm, d_v)
    plsc.subcore_barrier()

    cvec = plsc.load_gather(s_v, [jnp.full((LANES,), n_nodes, jnp.int32)])

    def chunk_base(c):
        return pl.multiple_of(wid * per_w + c * CHUNK, 32)

    def load_idx(c, j):
        base = chunk_base(c)
        pltpu.sync_copy(src_hbm.at[pl.ds(base, CHUNK)], srcs[j % 2])
        pltpu.sync_copy(dst_hbm.at[pl.ds(base, CHUNK)], dsts[j % 3])

    def issue_gather(c):
        pltpu.async_copy(h_hbm.at[srcs[c % 2]], hbufs[c % 3], sgs[c % 3])

    def wait_gather(c):
        pltpu.make_async_copy(h_hbm.at[srcs[c % 2]], hbufs[c % 3],
                              sgs[c % 3]).wait()

    def issue_scatter(c):
        pltpu.async_copy(hbufs[c % 3], u_sh.at[dsts[c % 3]], sus[c % 3],
                         add=True)
        pltpu.async_copy(eevs[c % 2], den_sh.at[dsts[c % 3]], sus[c % 3],
                         add=True)

    def wait_scatter(c):
        pltpu.make_async_copy(hbufs[c % 3], u_sh.at[dsts[c % 3]],
                              sus[c % 3]).wait()
        pltpu.make_async_copy(eevs[c % 2], den_sh.at[dsts[c % 3]],
                              sus[c % 3]).wait()

    def compute(c, j):
        base = chunk_base(c)
        srcv, dstv = srcs[j % 2], dsts[j % 3]
        hrows, eev = hbufs[j % 3], eevs[j % 2]
        for g in range(CHUNK // LANES):
            sidx = srcv[pl.ds(g * LANES, LANES)]
            didx = dstv[pl.ds(g * LANES, LANES)]
            e = plsc.load_gather(s_v, [sidx]) + plsc.load_gather(d_v, [didx])
            e = jnp.where(e > 0, e, NEG_SLOPE * e) - cvec
            gi = base + g * LANES + lax.iota(jnp.int32, LANES)
            eev[pl.ds(g * LANES, LANES)] = jnp.where(
                gi < n_total, jnp.exp(e), 0.0)

        def edge_body(k, carry):
            kidx = jnp.zeros((LANES,), jnp.int32) + k
            spl = plsc.load_gather(eev, [kidx])
            for cc in range(8):
                hrows[k, pl.ds(cc * LANES, LANES)] = (
                    hrows[k, pl.ds(cc * LANES, LANES)] * spl)
            return carry

        lax.fori_loop(0, CHUNK, edge_body, 0)

    def step(c, j, first_ring):
        if not (first_ring and j < 2):
            wait_scatter(j - 2)
        load_idx(c + 1, j + 1)
        issue_gather(j + 1)
        wait_gather(j)
        compute(c, j)
        issue_scatter(j)

    load_idx(0, 0)
    issue_gather(0)
    for j in range(RING):
        step(j, j, True)

    def ring_body(r, carry):
        c0 = r * RING
        for j in range(RING):
            step(c0 + j, j, False)
        return carry

    lax.fori_loop(1, n_chunks // RING, ring_body, 0)

    nl = n_chunks
    wait_gather(nl)
    wait_scatter(nl - 2)
    wait_scatter(nl - 1)
    plsc.subcore_barrier()

    @pl.when(sid == 0)
    def _():
        pltpu.sync_copy(u_sh, out_hbm.at[cid])
        pltpu.sync_copy(den_sh, den_hbm.at[cid])


def kernel(x, edge_index, W, att_src, att_dst, bias):
    n, _ = x.shape
    dout = W.shape[1]
    e = edge_index.shape[1]
    n_total = e + n
    n_chunks = -(-n_total // (NW * CHUNK))
    n_chunks = -(-n_chunks // RING) * RING
    per_w = n_chunks * CHUNK
    epad = NW * per_w + CHUNK
    n_ext = n + LANES

    loops = jnp.arange(n, dtype=edge_index.dtype)
    padz = jnp.zeros((epad - n_total,), edge_index.dtype)
    src = jnp.concatenate([edge_index[0], loops, padz])
    dst = jnp.concatenate([edge_index[1], loops, padz])

    h, s_ext, d_ext = pl.pallas_call(
        _prep_body,
        out_shape=(
            jax.ShapeDtypeStruct((n, dout), jnp.float32),
            jax.ShapeDtypeStruct((n_ext,), jnp.float32),
            jax.ShapeDtypeStruct((n_ext,), jnp.float32),
        ),
    )(x, W, att_src, att_dst)

    zero = jnp.zeros((n, dout), jnp.float32)
    zden = jnp.zeros((n,), jnp.float32)

    mesh = plsc.VectorSubcoreMesh(
        core_axis_name="c", subcore_axis_name="s",
        num_cores=NC, num_subcores=NS)
    cp = pltpu.CompilerParams()
    if "needs_layout_passes" in pltpu.CompilerParams.__dataclass_fields__:
        cp = dataclasses.replace(cp, needs_layout_passes=False)
    sc_fn = pl.kernel(
        functools.partial(_sc_body, n, n_total, per_w, n_chunks),
        out_type=(jax.ShapeDtypeStruct((NC, n, dout), jnp.float32),
                  jax.ShapeDtypeStruct((NC, n), jnp.float32)),
        mesh=mesh,
        compiler_params=cp,
        scratch_types=[
            pltpu.VMEM_SHARED((n, dout), jnp.float32),
            pltpu.VMEM_SHARED((n,), jnp.float32),
            pltpu.VMEM((n + LANES,), jnp.float32),
            pltpu.VMEM((n + LANES,), jnp.float32),
            pltpu.VMEM((CHUNK,), jnp.int32),
            pltpu.VMEM((CHUNK,), jnp.int32),
            pltpu.VMEM((CHUNK,), jnp.int32),
            pltpu.VMEM((CHUNK,), jnp.int32),
            pltpu.VMEM((CHUNK,), jnp.int32),
            pltpu.VMEM((CHUNK, 128), jnp.float32),
            pltpu.VMEM((CHUNK, 128), jnp.float32),
            pltpu.VMEM((CHUNK, 128), jnp.float32),
            pltpu.VMEM((CHUNK,), jnp.float32),
            pltpu.VMEM((CHUNK,), jnp.float32),
            pltpu.SemaphoreType.DMA,
            pltpu.SemaphoreType.DMA,
            pltpu.SemaphoreType.DMA,
            pltpu.SemaphoreType.DMA,
            pltpu.SemaphoreType.DMA,
            pltpu.SemaphoreType.DMA,
        ],
    )
    upart, dpart = sc_fn(h, s_ext, d_ext, src, dst, zero, zden)

    out = pl.pallas_call(
        _final_body,
        out_shape=jax.ShapeDtypeStruct((n, dout), jnp.float32),
    )(upart, dpart, bias)
    return out


# speedup vs baseline: 1.9306x; 1.0805x over previous
import dataclasses
import functools

import jax
import jax.numpy as jnp
from jax import lax
from jax.experimental import pallas as pl
from jax.experimental.pallas import tpu as pltpu
from jax.experimental.pallas import tpu_sc as plsc

NEG_SLOPE = 0.2
LANES = 16
CHUNK = 64
NC, NS = 2, 16
NW = NC * NS
RING = 6


def _prep_body(x_ref, w_ref, asrc_ref, adst_ref, h_ref, s_ref, d_ref):
    n = x_ref.shape[0]
    n_ext = s_ref.shape[0]
    h = jnp.dot(x_ref[...], w_ref[...], preferred_element_type=jnp.float32)
    h_ref[...] = h
    s = jnp.dot(h, asrc_ref[...].reshape(-1, 1),
                preferred_element_type=jnp.float32)[:, 0]
    d = jnp.dot(h, adst_ref[...].reshape(-1, 1),
                preferred_element_type=jnp.float32)[:, 0]
    c = jnp.max(s) + jnp.max(d)
    c = jnp.where(c > 0, c, NEG_SLOPE * c)
    pad = jnp.zeros((n_ext - n,), jnp.float32)
    s_ext = jnp.concatenate([s, pad])
    d_ext = jnp.concatenate([d, pad])
    idx = lax.broadcasted_iota(jnp.int32, (n_ext,), 0)
    s_ref[...] = jnp.where(idx == n, c, s_ext)
    d_ref[...] = d_ext


def _final_body(u_ref, d_ref, bias_ref, out_ref):
    num = u_ref[0] + u_ref[1]
    den = d_ref[0] + d_ref[1]
    out_ref[...] = jnp.tanh(num / den[:, None] + bias_ref[...][None, :])


def _sc_body(n_nodes, n_total, per_w, n_chunks,
             h_hbm, s_hbm, d_hbm, edge_hbm, zero_hbm, zden_hbm,
             out_hbm, den_hbm,
             u_sh, den_sh, s_v, d_v,
             ev0, ev1, tv0, tv1, tv2, hb0, hb1, hb2, ee0, ee1,
             sg0, sg1, sg2, su0, su1, su2):
    cid = lax.axis_index("c")
    sid = lax.axis_index("s")
    wid = sid * NC + cid
    rows_per_tile = (n_nodes // NS) // 8 * 8
    rem_base = rows_per_tile * NS
    rem = n_nodes - rem_base
    edvs = (ev0, ev1)
    dsts = (tv0, tv1, tv2)
    hbufs = (hb0, hb1, hb2)
    eevs = (ee0, ee1)
    sgs = (sg0, sg1, sg2)
    sus = (su0, su1, su2)

    off = pl.multiple_of(sid * rows_per_tile, 8)
    pltpu.sync_copy(zero_hbm.at[pl.ds(off, rows_per_tile)],
                    u_sh.at[pl.ds(off, rows_per_tile)])
    if rem:
        @pl.when(sid == 0)
        def _():
            pltpu.sync_copy(zero_hbm.at[pl.ds(rem_base, rem)],
                            u_sh.at[pl.ds(rem_base, rem)])

    @pl.when(sid == 0)
    def _():
        pltpu.sync_copy(zden_hbm, den_sh)

    pltpu.sync_copy(s_hbm, s_v)
    pltpu.sync_copy(d_hbm, d_v)
    plsc.subcore_barrier()

    cvec = plsc.load_gather(s_v, [jnp.full((LANES,), n_nodes, jnp.int32)])

    def chunk_base(c):
        return pl.multiple_of(wid * per_w + c * CHUNK, 32)

    def chunk_row(c):
        return wid * n_chunks + c

    def load_idx(c, j):
        pltpu.sync_copy(edge_hbm.at[chunk_row(c)], edvs[j % 2])

    def issue_gather(c):
        pltpu.async_copy(h_hbm.at[edvs[c % 2].at[0]], hbufs[c % 3],
                         sgs[c % 3])

    def wait_gather(c):
        pltpu.make_async_copy(h_hbm.at[edvs[c % 2].at[0]], hbufs[c % 3],
                              sgs[c % 3]).wait()

    def issue_scatter(c):
        pltpu.async_copy(hbufs[c % 3], u_sh.at[dsts[c % 3]], sus[c % 3],
                         add=True)
        pltpu.async_copy(eevs[c % 2], den_sh.at[dsts[c % 3]], sus[c % 3],
                         add=True)

    def wait_scatter(c):
        pltpu.make_async_copy(hbufs[c % 3], u_sh.at[dsts[c % 3]],
                              sus[c % 3]).wait()
        pltpu.make_async_copy(eevs[c % 2], den_sh.at[dsts[c % 3]],
                              sus[c % 3]).wait()

    def compute(c, j):
        base = chunk_base(c)
        edv, dstv = edvs[j % 2], dsts[j % 3]
        hrows, eev = hbufs[j % 3], eevs[j % 2]
        for g in range(CHUNK // LANES):
            sidx = edv[0, pl.ds(g * LANES, LANES)]
            didx = edv[1, pl.ds(g * LANES, LANES)]
            dstv[pl.ds(g * LANES, LANES)] = didx
            e = plsc.load_gather(s_v, [sidx]) + plsc.load_gather(d_v, [didx])
            e = jnp.where(e > 0, e, NEG_SLOPE * e) - cvec
            gi = base + g * LANES + lax.iota(jnp.int32, LANES)
            eev[pl.ds(g * LANES, LANES)] = jnp.where(
                gi < n_total, jnp.exp(e), 0.0)

        def edge_body(k, carry):
            kidx = jnp.zeros((LANES,), jnp.int32) + k
            spl = plsc.load_gather(eev, [kidx])
            for cc in range(8):
                hrows[k, pl.ds(cc * LANES, LANES)] = (
                    hrows[k, pl.ds(cc * LANES, LANES)] * spl)
            return carry

        lax.fori_loop(0, CHUNK, edge_body, 0)

    def step(c, j, first_ring):
        if not (first_ring and j < 2):
            wait_scatter(j - 2)
        load_idx(c + 1, j + 1)
        issue_gather(j + 1)
        wait_gather(j)
        compute(c, j)
        issue_scatter(j)

    load_idx(0, 0)
    issue_gather(0)
    for j in range(RING):
        step(j, j, True)

    def ring_body(r, carry):
        c0 = r * RING
        for j in range(RING):
            step(c0 + j, j, False)
        return carry

    lax.fori_loop(1, n_chunks // RING, ring_body, 0)

    nl = n_chunks
    wait_gather(nl)
    wait_scatter(nl - 2)
    wait_scatter(nl - 1)
    plsc.subcore_barrier()

    @pl.when(sid == 0)
    def _():
        pltpu.sync_copy(u_sh, out_hbm.at[cid])
        pltpu.sync_copy(den_sh, den_hbm.at[cid])


def kernel(x, edge_index, W, att_src, att_dst, bias):
    n, _ = x.shape
    dout = W.shape[1]
    e = edge_index.shape[1]
    n_total = e + n
    n_chunks = -(-n_total // (NW * CHUNK))
    n_chunks = -(-n_chunks // RING) * RING
    per_w = n_chunks * CHUNK
    epad = NW * per_w + CHUNK
    n_ext = n + LANES

    loops = jnp.arange(n, dtype=edge_index.dtype)
    padz = jnp.zeros((epad - n_total,), edge_index.dtype)
    src = jnp.concatenate([edge_index[0], loops, padz])
    dst = jnp.concatenate([edge_index[1], loops, padz])
    edges = jnp.stack([src.reshape(-1, CHUNK), dst.reshape(-1, CHUNK)],
                      axis=1)

    h, s_ext, d_ext = pl.pallas_call(
        _prep_body,
        out_shape=(
            jax.ShapeDtypeStruct((n, dout), jnp.float32),
            jax.ShapeDtypeStruct((n_ext,), jnp.float32),
            jax.ShapeDtypeStruct((n_ext,), jnp.float32),
        ),
    )(x, W, att_src, att_dst)

    zero = jnp.zeros((n, dout), jnp.float32)
    zden = jnp.zeros((n,), jnp.float32)

    mesh = plsc.VectorSubcoreMesh(
        core_axis_name="c", subcore_axis_name="s",
        num_cores=NC, num_subcores=NS)
    cp = pltpu.CompilerParams()
    if "needs_layout_passes" in pltpu.CompilerParams.__dataclass_fields__:
        cp = dataclasses.replace(cp, needs_layout_passes=False)
    sc_fn = pl.kernel(
        functools.partial(_sc_body, n, n_total, per_w, n_chunks),
        out_type=(jax.ShapeDtypeStruct((NC, n, dout), jnp.float32),
                  jax.ShapeDtypeStruct((NC, n), jnp.float32)),
        mesh=mesh,
        compiler_params=cp,
        scratch_types=[
            pltpu.VMEM_SHARED((n, dout), jnp.float32),
            pltpu.VMEM_SHARED((n,), jnp.float32),
            pltpu.VMEM((n + LANES,), jnp.float32),
            pltpu.VMEM((n + LANES,), jnp.float32),
            pltpu.VMEM((2, CHUNK), jnp.int32),
            pltpu.VMEM((2, CHUNK), jnp.int32),
            pltpu.VMEM((CHUNK,), jnp.int32),
            pltpu.VMEM((CHUNK,), jnp.int32),
            pltpu.VMEM((CHUNK,), jnp.int32),
            pltpu.VMEM((CHUNK, 128), jnp.float32),
            pltpu.VMEM((CHUNK, 128), jnp.float32),
            pltpu.VMEM((CHUNK, 128), jnp.float32),
            pltpu.VMEM((CHUNK,), jnp.float32),
            pltpu.VMEM((CHUNK,), jnp.float32),
            pltpu.SemaphoreType.DMA,
            pltpu.SemaphoreType.DMA,
            pltpu.SemaphoreType.DMA,
            pltpu.SemaphoreType.DMA,
            pltpu.SemaphoreType.DMA,
            pltpu.SemaphoreType.DMA,
        ],
    )
    upart, dpart = sc_fn(h, s_ext, d_ext, edges, zero, zden)

    out = pl.pallas_call(
        _final_body,
        out_shape=jax.ShapeDtypeStruct((n, dout), jnp.float32),
    )(upart, dpart, bias)
    return out


# speedup vs baseline: 2.2118x; 1.1456x over previous
import dataclasses
import functools

import jax
import jax.numpy as jnp
from jax import lax
from jax.experimental import pallas as pl
from jax.experimental.pallas import tpu as pltpu
from jax.experimental.pallas import tpu_sc as plsc

NEG_SLOPE = 0.2
LANES = 16
CHUNK = 64
NC, NS = 2, 16
NW = NC * NS
RING = 6


def _prep_body(x_ref, w_ref, asrc_ref, adst_ref, h_ref, s_ref, d_ref):
    n = x_ref.shape[0]
    n_ext = s_ref.shape[0]
    h = jnp.dot(x_ref[...], w_ref[...], preferred_element_type=jnp.float32)
    h_ref[...] = h
    s = jnp.dot(h, asrc_ref[...].reshape(-1, 1),
                preferred_element_type=jnp.float32)[:, 0]
    d = jnp.dot(h, adst_ref[...].reshape(-1, 1),
                preferred_element_type=jnp.float32)[:, 0]
    c = jnp.max(s) + jnp.max(d)
    c = jnp.where(c > 0, c, NEG_SLOPE * c)
    pad = jnp.zeros((n_ext - n,), jnp.float32)
    s_ext = jnp.concatenate([s, pad])
    d_ext = jnp.concatenate([d, pad])
    idx = lax.broadcasted_iota(jnp.int32, (n_ext,), 0)
    s_ref[...] = jnp.where(idx == n, c, s_ext)
    d_ref[...] = d_ext


def _final_body(u_ref, d_ref, bias_ref, out_ref):
    num = u_ref[0] + u_ref[1]
    den = d_ref[0] + d_ref[1]
    out_ref[...] = jnp.tanh(num / den[:, None] + bias_ref[...][None, :])


def _sc_body(n_nodes, n_total, per_w, n_chunks,
             h_hbm, s_hbm, d_hbm, edge_hbm, zero_hbm, zden_hbm,
             out_hbm, den_hbm,
             u_sh, den_sh, s_v, d_v,
             ev0, ev1, tv0, tv1, tv2, hb0, hb1, hb2, ee0, ee1,
             sg0, sg1, sg2, su0, su1, su2):
    cid = lax.axis_index("c")
    sid = lax.axis_index("s")
    wid = sid * NC + cid
    rows_per_tile = (n_nodes // NS) // 8 * 8
    rem_base = rows_per_tile * NS
    rem = n_nodes - rem_base
    edvs = (ev0, ev1)
    dsts = (tv0, tv1, tv2)
    hbufs = (hb0, hb1, hb2)
    eevs = (ee0, ee1)
    sgs = (sg0, sg1, sg2)
    sus = (su0, su1, su2)

    off = pl.multiple_of(sid * rows_per_tile, 8)
    pltpu.sync_copy(zero_hbm.at[pl.ds(off, rows_per_tile)],
                    u_sh.at[pl.ds(off, rows_per_tile)])
    if rem:
        @pl.when(sid == 0)
        def _():
            pltpu.sync_copy(zero_hbm.at[pl.ds(rem_base, rem)],
                            u_sh.at[pl.ds(rem_base, rem)])

    @pl.when(sid == 0)
    def _():
        pltpu.sync_copy(zden_hbm, den_sh)

    pltpu.sync_copy(s_hbm, s_v)
    pltpu.sync_copy(d_hbm, d_v)
    plsc.subcore_barrier()

    cvec = plsc.load_gather(s_v, [jnp.full((LANES,), n_nodes, jnp.int32)])

    def chunk_base(c):
        return pl.multiple_of(wid * per_w + c * CHUNK, 32)

    def chunk_row(c):
        return wid * n_chunks + c

    def load_idx(c, j):
        pltpu.sync_copy(edge_hbm.at[chunk_row(c)], edvs[j % 2])

    def issue_gather(c):
        pltpu.async_copy(h_hbm.at[edvs[c % 2].at[0]], hbufs[c % 3],
                         sgs[c % 3])

    def wait_gather(c):
        pltpu.make_async_copy(h_hbm.at[edvs[c % 2].at[0]], hbufs[c % 3],
                              sgs[c % 3]).wait()

    def issue_scatter(c):
        pltpu.async_copy(hbufs[c % 3], u_sh.at[dsts[c % 3]], sus[c % 3],
                         add=True)
        pltpu.async_copy(eevs[c % 2], den_sh.at[dsts[c % 3]], sus[c % 3],
                         add=True)

    def wait_scatter(c):
        pltpu.make_async_copy(hbufs[c % 3], u_sh.at[dsts[c % 3]],
                              sus[c % 3]).wait()
        pltpu.make_async_copy(eevs[c % 2], den_sh.at[dsts[c % 3]],
                              sus[c % 3]).wait()

    def compute(c, j):
        base = chunk_base(c)
        edv, dstv = edvs[j % 2], dsts[j % 3]
        hrows, eev = hbufs[j % 3], eevs[j % 2]
        for g in range(CHUNK // LANES):
            sidx = edv[0, pl.ds(g * LANES, LANES)]
            didx = edv[1, pl.ds(g * LANES, LANES)]
            dstv[pl.ds(g * LANES, LANES)] = didx
            e = plsc.load_gather(s_v, [sidx]) + plsc.load_gather(d_v, [didx])
            e = jnp.where(e > 0, e, NEG_SLOPE * e) - cvec
            gi = base + g * LANES + lax.iota(jnp.int32, LANES)
            eev[pl.ds(g * LANES, LANES)] = jnp.where(
                gi < n_total, jnp.exp(e), 0.0)

        @functools.partial(plsc.parallel_loop, 0, CHUNK, unroll=4)
        def edge_body(k):
            kidx = jnp.zeros((LANES,), jnp.int32) + k
            spl = plsc.load_gather(eev, [kidx])
            for cc in range(8):
                hrows[k, pl.ds(cc * LANES, LANES)] = (
                    hrows[k, pl.ds(cc * LANES, LANES)] * spl)

    def step(c, j, first_ring):
        if not (first_ring and j < 2):
            wait_scatter(j - 2)
        load_idx(c + 1, j + 1)
        issue_gather(j + 1)
        wait_gather(j)
        compute(c, j)
        issue_scatter(j)

    load_idx(0, 0)
    issue_gather(0)
    for j in range(RING):
        step(j, j, True)

    def ring_body(r, carry):
        c0 = r * RING
        for j in range(RING):
            step(c0 + j, j, False)
        return carry

    lax.fori_loop(1, n_chunks // RING, ring_body, 0)

    nl = n_chunks
    wait_gather(nl)
    wait_scatter(nl - 2)
    wait_scatter(nl - 1)
    plsc.subcore_barrier()

    @pl.when(sid == 0)
    def _():
        pltpu.sync_copy(u_sh, out_hbm.at[cid])
        pltpu.sync_copy(den_sh, den_hbm.at[cid])


def kernel(x, edge_index, W, att_src, att_dst, bias):
    n, _ = x.shape
    dout = W.shape[1]
    e = edge_index.shape[1]
    n_total = e + n
    n_chunks = -(-n_total // (NW * CHUNK))
    n_chunks = -(-n_chunks // RING) * RING
    per_w = n_chunks * CHUNK
    epad = NW * per_w + CHUNK
    n_ext = n + LANES

    loops = jnp.arange(n, dtype=edge_index.dtype)
    padz = jnp.zeros((epad - n_total,), edge_index.dtype)
    src = jnp.concatenate([edge_index[0], loops, padz])
    dst = jnp.concatenate([edge_index[1], loops, padz])
    edges = jnp.stack([src.reshape(-1, CHUNK), dst.reshape(-1, CHUNK)],
                      axis=1)

    h, s_ext, d_ext = pl.pallas_call(
        _prep_body,
        out_shape=(
            jax.ShapeDtypeStruct((n, dout), jnp.float32),
            jax.ShapeDtypeStruct((n_ext,), jnp.float32),
            jax.ShapeDtypeStruct((n_ext,), jnp.float32),
        ),
    )(x, W, att_src, att_dst)

    zero = jnp.zeros((n, dout), jnp.float32)
    zden = jnp.zeros((n,), jnp.float32)

    mesh = plsc.VectorSubcoreMesh(
        core_axis_name="c", subcore_axis_name="s",
        num_cores=NC, num_subcores=NS)
    cp = pltpu.CompilerParams()
    if "needs_layout_passes" in pltpu.CompilerParams.__dataclass_fields__:
        cp = dataclasses.replace(cp, needs_layout_passes=False)
    sc_fn = pl.kernel(
        functools.partial(_sc_body, n, n_total, per_w, n_chunks),
        out_type=(jax.ShapeDtypeStruct((NC, n, dout), jnp.float32),
                  jax.ShapeDtypeStruct((NC, n), jnp.float32)),
        mesh=mesh,
        compiler_params=cp,
        scratch_types=[
            pltpu.VMEM_SHARED((n, dout), jnp.float32),
            pltpu.VMEM_SHARED((n,), jnp.float32),
            pltpu.VMEM((n + LANES,), jnp.float32),
            pltpu.VMEM((n + LANES,), jnp.float32),
            pltpu.VMEM((2, CHUNK), jnp.int32),
            pltpu.VMEM((2, CHUNK), jnp.int32),
            pltpu.VMEM((CHUNK,), jnp.int32),
            pltpu.VMEM((CHUNK,), jnp.int32),
            pltpu.VMEM((CHUNK,), jnp.int32),
            pltpu.VMEM((CHUNK, 128), jnp.float32),
            pltpu.VMEM((CHUNK, 128), jnp.float32),
            pltpu.VMEM((CHUNK, 128), jnp.float32),
            pltpu.VMEM((CHUNK,), jnp.float32),
            pltpu.VMEM((CHUNK,), jnp.float32),
            pltpu.SemaphoreType.DMA,
            pltpu.SemaphoreType.DMA,
            pltpu.SemaphoreType.DMA,
            pltpu.SemaphoreType.DMA,
            pltpu.SemaphoreType.DMA,
            pltpu.SemaphoreType.DMA,
        ],
    )
    upart, dpart = sc_fn(h, s_ext, d_ext, edges, zero, zden)

    out = pl.pallas_call(
        _final_body,
        out_shape=jax.ShapeDtypeStruct((n, dout), jnp.float32),
    )(upart, dpart, bias)
    return out
